# baseline (speedup 1.0000x reference)
import jax
import jax.numpy as jnp
from jax.experimental import pallas as pl
from jax.experimental.pallas import tpu as pltpu

_ROWS = 1024
_LBL = 512
_JB = 128
_GRID = _ROWS // _JB


def _loss_body(p_ref, s_ref, a_ref, out_ref, cand_ref, acc_ref, rs_ref):
    j = pl.program_id(0)

    @pl.when(j == 0)
    def _init():
        inv_l = jnp.float32(1.0 / _LBL)
        cand_ref[...] = (
            jnp.dot(p_ref[...], a_ref[...], preferred_element_type=jnp.float32)
            * inv_l
        )

    s = s_ref[...]
    c_j = cand_ref[pl.ds(j * _JB, _JB), :]
    partial = jnp.dot(s, c_j, preferred_element_type=jnp.float32)
    rs_part = jnp.sum(s, axis=1, keepdims=True)

    @pl.when(j == 0)
    def _first():
        acc_ref[...] = partial
        rs_ref[...] = rs_part

    @pl.when(j > 0)
    def _rest():
        acc_ref[...] += partial
        rs_ref[...] += rs_part

    @pl.when(j == _GRID - 1)
    def _emit():
        diff = p_ref[...] - acc_ref[...]
        sq = jnp.sum(diff * diff, axis=1)
        norms = jnp.sqrt(sq)
        mask = rs_ref[...][:, 0] != 0
        cnt = jnp.sum(mask.astype(jnp.float32))
        total = jnp.sum(jnp.where(mask, norms, jnp.float32(0.0)))
        out_ref[...] = jnp.reshape(total / cnt, (1, 1))


def kernel(predicts, similarities, adjList):
    out = pl.pallas_call(
        _loss_body,
        grid=(_GRID,),
        in_specs=[
            pl.BlockSpec((_ROWS, _LBL), lambda j: (0, 0)),
            pl.BlockSpec((_ROWS, _JB), lambda j: (0, j)),
            pl.BlockSpec((_LBL, _LBL), lambda j: (0, 0)),
        ],
        out_specs=pl.BlockSpec((1, 1), lambda j: (0, 0)),
        out_shape=jax.ShapeDtypeStruct((1, 1), jnp.float32),
        scratch_shapes=[
            pltpu.VMEM((_ROWS, _LBL), jnp.float32),
            pltpu.VMEM((_ROWS, _LBL), jnp.float32),
            pltpu.VMEM((_ROWS, 1), jnp.float32),
        ],
    )(predicts, similarities, adjList)
    return out[0, 0]
